# baseline (device time: 12423 ns/iter reference)
import jax
import jax.numpy as jnp
from jax import lax
from jax.experimental import pallas as pl
from jax.experimental.pallas import tpu as pltpu

N_DEV = 8


def kernel(x, w_mat):
    m_total, k_shard = x.shape
    k_total, n = w_mat.shape
    m_blk = m_total // N_DEV

    def body(x_ref, w_ref, out_ref, xb_ref, xg_ref, wb_ref,
             send_sems, recv_sems):
        my = lax.axis_index("i")

        barrier = pltpu.get_barrier_semaphore()
        for d in range(1, N_DEV):
            pl.semaphore_signal(
                barrier, inc=1,
                device_id=((my + d) % N_DEV,),
                device_id_type=pl.DeviceIdType.MESH,
            )
        pl.semaphore_wait(barrier, N_DEV - 1)

        xb_ref[...] = x_ref[...].astype(jnp.bfloat16)

        sends = []
        for d in range(1, N_DEV):
            t = (my + d) % N_DEV
            rdma = pltpu.make_async_remote_copy(
                src_ref=xb_ref.at[pl.ds(t * m_blk, m_blk), :],
                dst_ref=xg_ref.at[:, pl.ds(my * m_blk, m_blk)],
                send_sem=send_sems.at[d],
                recv_sem=recv_sems.at[d],
                device_id=(t,),
                device_id_type=pl.DeviceIdType.MESH,
            )
            rdma.start()
            sends.append(rdma)

        xg_ref[:, pl.ds(my * m_blk, m_blk)] = xb_ref[pl.ds(my * m_blk, m_blk), :]

        wb_ref[...] = w_ref[...].astype(jnp.bfloat16)

        for d in range(1, N_DEV):
            s = (my - d) % N_DEV
            recv = pltpu.make_async_remote_copy(
                src_ref=xb_ref.at[pl.ds(s * m_blk, m_blk), :],
                dst_ref=xg_ref.at[:, pl.ds(s * m_blk, m_blk)],
                send_sem=send_sems.at[d],
                recv_sem=recv_sems.at[d],
                device_id=(s,),
                device_id_type=pl.DeviceIdType.MESH,
            )
            recv.wait_recv()

        out_ref[...] = jnp.dot(
            xg_ref[...], wb_ref[...], preferred_element_type=jnp.float32
        )

        for rdma in sends:
            rdma.wait_send()

    return pl.pallas_call(
        body,
        out_shape=jax.ShapeDtypeStruct((m_blk, n), jnp.float32),
        in_specs=[
            pl.BlockSpec(memory_space=pltpu.VMEM),
            pl.BlockSpec(memory_space=pltpu.VMEM),
        ],
        out_specs=pl.BlockSpec(memory_space=pltpu.VMEM),
        scratch_shapes=[
            pltpu.VMEM((m_total, k_shard), jnp.bfloat16),
            pltpu.VMEM((m_blk, k_total), jnp.bfloat16),
            pltpu.VMEM((k_total, n), jnp.bfloat16),
            pltpu.SemaphoreType.DMA((N_DEV,)),
            pltpu.SemaphoreType.DMA((N_DEV,)),
        ],
        compiler_params=pltpu.CompilerParams(collective_id=0),
    )(x, w_mat)


# device time: 10252 ns/iter; 1.2118x vs baseline; 1.2118x over previous
import jax
import jax.numpy as jnp
from jax import lax
from jax.experimental import pallas as pl
from jax.experimental.pallas import tpu as pltpu

N_DEV = 8
W_CHUNKS = 4


def kernel(x, w_mat):
    m_total, k_shard = x.shape
    k_total, n = w_mat.shape
    m_blk = m_total // N_DEV
    k_chunk = k_total // W_CHUNKS

    def body(x_ref, w_ref, out_ref, xv_ref, wf_ref, xb_ref, xg_ref, wb_ref,
             y_ref, send_sems, recv_sems, xcp_sem, wcp_sems, out_sem):
        my = lax.axis_index("i")

        xcp = pltpu.make_async_copy(x_ref, xv_ref, xcp_sem)
        xcp.start()
        wcps = []
        for c in range(W_CHUNKS):
            wcp = pltpu.make_async_copy(
                w_ref.at[pl.ds(c * k_chunk, k_chunk), :],
                wf_ref.at[pl.ds(c * k_chunk, k_chunk), :],
                wcp_sems.at[c],
            )
            wcp.start()
            wcps.append(wcp)

        barrier = pltpu.get_barrier_semaphore()
        for d in range(1, N_DEV):
            pl.semaphore_signal(
                barrier, inc=1,
                device_id=((my + d) % N_DEV,),
                device_id_type=pl.DeviceIdType.MESH,
            )

        xcp.wait()
        xb_ref[...] = xv_ref[...].astype(jnp.bfloat16)

        pl.semaphore_wait(barrier, N_DEV - 1)

        sends = []
        for d in range(1, N_DEV):
            t = (my + d) % N_DEV
            rdma = pltpu.make_async_remote_copy(
                src_ref=xb_ref.at[pl.ds(t * m_blk, m_blk), :],
                dst_ref=xg_ref.at[:, pl.ds(my * m_blk, m_blk)],
                send_sem=send_sems.at[d],
                recv_sem=recv_sems.at[d],
                device_id=(t,),
                device_id_type=pl.DeviceIdType.MESH,
            )
            rdma.start()
            sends.append(rdma)

        xg_ref[:, pl.ds(my * m_blk, m_blk)] = xb_ref[pl.ds(my * m_blk, m_blk), :]

        for c in range(W_CHUNKS):
            wcps[c].wait()
            wb_ref[pl.ds(c * k_chunk, k_chunk), :] = (
                wf_ref[pl.ds(c * k_chunk, k_chunk), :].astype(jnp.bfloat16)
            )

        for d in range(1, N_DEV):
            s = (my - d) % N_DEV
            recv = pltpu.make_async_remote_copy(
                src_ref=xb_ref.at[pl.ds(s * m_blk, m_blk), :],
                dst_ref=xg_ref.at[:, pl.ds(s * m_blk, m_blk)],
                send_sem=send_sems.at[d],
                recv_sem=recv_sems.at[d],
                device_id=(s,),
                device_id_type=pl.DeviceIdType.MESH,
            )
            recv.wait_recv()

        y_ref[...] = jnp.dot(
            xg_ref[...], wb_ref[...], preferred_element_type=jnp.float32
        )

        ocp = pltpu.make_async_copy(y_ref, out_ref, out_sem)
        ocp.start()
        ocp.wait()

        for rdma in sends:
            rdma.wait_send()

    x = pltpu.with_memory_space_constraint(x, pltpu.MemorySpace.HBM)
    w_mat = pltpu.with_memory_space_constraint(w_mat, pltpu.MemorySpace.HBM)

    return pl.pallas_call(
        body,
        out_shape=jax.ShapeDtypeStruct((m_blk, n), jnp.float32),
        in_specs=[
            pl.BlockSpec(memory_space=pltpu.MemorySpace.HBM),
            pl.BlockSpec(memory_space=pltpu.MemorySpace.HBM),
        ],
        out_specs=pl.BlockSpec(memory_space=pltpu.MemorySpace.HBM),
        scratch_shapes=[
            pltpu.VMEM((m_total, k_shard), jnp.float32),
            pltpu.VMEM((k_total, n), jnp.float32),
            pltpu.VMEM((m_total, k_shard), jnp.bfloat16),
            pltpu.VMEM((m_blk, k_total), jnp.bfloat16),
            pltpu.VMEM((k_total, n), jnp.bfloat16),
            pltpu.VMEM((m_blk, n), jnp.float32),
            pltpu.SemaphoreType.DMA((N_DEV,)),
            pltpu.SemaphoreType.DMA((N_DEV,)),
            pltpu.SemaphoreType.DMA,
            pltpu.SemaphoreType.DMA((W_CHUNKS,)),
            pltpu.SemaphoreType.DMA,
        ],
        compiler_params=pltpu.CompilerParams(collective_id=0),
    )(x, w_mat)
